# Initial kernel scaffold; baseline (speedup 1.0000x reference)
import jax
import jax.numpy as jnp
from jax.experimental import pallas as pl


def kernel(x, W_enc, b_enc, W_dec, b_dec):
    raise NotImplementedError("write your pallas kernel here")


# speedup vs baseline: 3.6203x; 3.6203x over previous
import functools

import jax
import jax.numpy as jnp
from jax import lax
from jax.experimental import pallas as pl
from jax.experimental.pallas import tpu as pltpu
from jax.experimental.pallas import tpu_sc as plsc

D_IN = 768
HIDDEN = 12288
K = 32
B = 8192
GROUP = 16
NGROUP = HIDDEN // GROUP

RB = 256
CB = 2048

NC = 2
NS = 16
NW = NC * NS
ROWS_PER_W = B // NW


def _enc_body(x_ref, w_ref, benc_ref, bdec_ref, acts_ref, summ_ref):
    xb = x_ref[...] - bdec_ref[...]
    acc = lax.dot_general(
        xb, w_ref[...],
        dimension_numbers=(((1,), (1,)), ((), ())),
        preferred_element_type=jnp.float32,
    )
    acts = jnp.maximum(acc + benc_ref[...], 0.0)
    acts_ref[...] = acts
    summ_ref[...] = jnp.max(acts.reshape(RB, CB // GROUP, GROUP), axis=-1)


def _encode(x, W_enc, b_enc, b_dec):
    grid = (HIDDEN // CB, B // RB)
    return pl.pallas_call(
        _enc_body,
        grid=grid,
        in_specs=[
            pl.BlockSpec((RB, D_IN), lambda j, i: (i, 0)),
            pl.BlockSpec((CB, D_IN), lambda j, i: (j, 0)),
            pl.BlockSpec((1, CB), lambda j, i: (0, j)),
            pl.BlockSpec((1, D_IN), lambda j, i: (0, 0)),
        ],
        out_specs=[
            pl.BlockSpec((RB, CB), lambda j, i: (i, j)),
            pl.BlockSpec((RB, CB // GROUP), lambda j, i: (i, j)),
        ],
        out_shape=[
            jax.ShapeDtypeStruct((B, HIDDEN), jnp.float32),
            jax.ShapeDtypeStruct((B, NGROUP), jnp.float32),
        ],
        compiler_params=pltpu.CompilerParams(
            dimension_semantics=("arbitrary", "arbitrary"),
        ),
    )(x, W_enc, b_enc.reshape(1, HIDDEN), b_dec.reshape(1, D_IN))


def _bcast(v, j):
    dnums = lax.GatherDimensionNumbers(
        offset_dims=(), collapsed_slice_dims=(0,), start_index_map=(0,))
    return lax.gather(v, jnp.full((16, 1), j, jnp.int32), dnums,
                      slice_sizes=(1,),
                      mode=lax.GatherScatterMode.PROMISE_IN_BOUNDS)


def _merge(state, ck, ci):
    t1k, t1i, t2k, t2i = state
    ckd, cid = plsc.sort_key_val(ck, ci, descending=True)
    m = t2k >= ckd
    h2k = jnp.where(m, t2k, ckd)
    h2i = jnp.where(m, t2i, cid)
    h2k, h2i = plsc.sort_key_val(h2k, h2i, descending=True)
    m2 = t1k >= h2k
    h1k = jnp.where(m2, t1k, h2k)
    h1i = jnp.where(m2, t1i, h2i)
    l1k = jnp.where(m2, h2k, t1k)
    l1i = jnp.where(m2, h2i, t1i)
    t1k, t1i = plsc.sort_key_val(h1k, h1i)
    t2k, t2i = plsc.sort_key_val(l1k, l1i)
    return (t1k, t1i, t2k, t2i)


def _sc_body(pa_ref, summ_ref, wdec_ref, bdec_ref, out_ref,
             s_v, cidx_v, cand_v, widx_v, wrows_v, bdec_v, acc_v,
             sem_c, sem_w):
    wid = lax.axis_index("s") * NC + lax.axis_index("c")
    row0 = wid * ROWS_PER_W
    pltpu.sync_copy(bdec_ref, bdec_v)
    lane = lax.iota(jnp.int32, 16)
    neg = jnp.full((16,), -1.0, jnp.float32)
    zero = jnp.zeros((16,), jnp.int32)

    def row_body(r, carry):
        row = row0 + r
        pltpu.sync_copy(summ_ref.at[row], s_v)

        def ph1(g, st):
            ck = s_v[pl.ds(g * GROUP, GROUP)]
            return _merge(st, ck, lane + g * GROUP)
        t1k, t1i, t2k, t2i = lax.fori_loop(0, NGROUP // GROUP, ph1,
                                           (neg, zero, neg, zero))

        cidx_v[pl.ds(0, 16)] = t1i + row * NGROUP
        cidx_v[pl.ds(16, 16)] = t2i + row * NGROUP
        pltpu.async_copy(pa_ref.at[cidx_v], cand_v, sem_c).wait()

        def ph2(j, st):
            st = _merge(st, cand_v[j, pl.ds(0, GROUP)],
                        _bcast(t1i, j) * GROUP + lane)
            st = _merge(st, cand_v[j + 16, pl.ds(0, GROUP)],
                        _bcast(t2i, j) * GROUP + lane)
            return st
        v1, i1, v2, i2 = lax.fori_loop(0, 16, ph2, (neg, zero, neg, zero))

        widx_v[pl.ds(0, 16)] = i1
        widx_v[pl.ds(16, 16)] = i2
        pltpu.async_copy(wdec_ref.at[widx_v], wrows_v, sem_w).wait()

        for c in range(D_IN // 16):
            acc_v[pl.ds(c * 16, 16)] = bdec_v[pl.ds(c * 16, 16)]

        def dec(k, vk_base):
            vk, base = vk_base
            v = _bcast(vk, k)
            for c in range(D_IN // 16):
                plsc.addupdate(acc_v.at[pl.ds(c * 16, 16)],
                               v * wrows_v[base + k, pl.ds(c * 16, 16)])
            return vk_base
        lax.fori_loop(0, 16, dec, (v1, 0))
        lax.fori_loop(0, 16, dec, (v2, 16))

        pltpu.sync_copy(acc_v, out_ref.at[row])
        return carry

    lax.fori_loop(0, ROWS_PER_W, row_body, 0)


def _decode_topk(pre_acts, summaries, W_dec, b_dec):
    pa_flat = pre_acts.reshape(B * NGROUP, GROUP)
    mesh = plsc.VectorSubcoreMesh(core_axis_name="c", subcore_axis_name="s")
    f = pl.kernel(
        _sc_body,
        out_type=jax.ShapeDtypeStruct((B, D_IN), jnp.float32),
        mesh=mesh,
        scratch_types=[
            pltpu.VMEM((NGROUP,), jnp.float32),
            pltpu.VMEM((K,), jnp.int32),
            pltpu.VMEM((K, GROUP), jnp.float32),
            pltpu.VMEM((K,), jnp.int32),
            pltpu.VMEM((K, D_IN), jnp.float32),
            pltpu.VMEM((D_IN,), jnp.float32),
            pltpu.VMEM((D_IN,), jnp.float32),
            pltpu.SemaphoreType.DMA,
            pltpu.SemaphoreType.DMA,
        ],
        compiler_params=pltpu.CompilerParams(
            needs_layout_passes=False, use_tc_tiling_on_sc=False),
    )
    return f(pa_flat, summaries, W_dec, b_dec)


def kernel(x, W_enc, b_enc, W_dec, b_dec):
    pre_acts, summaries = _encode(x, W_enc, b_enc, b_dec)
    return _decode_topk(pre_acts, summaries, W_dec, b_dec)


# speedup vs baseline: 4.3128x; 1.1913x over previous
import functools

import jax
import jax.numpy as jnp
from jax import lax
from jax.experimental import pallas as pl
from jax.experimental.pallas import tpu as pltpu
from jax.experimental.pallas import tpu_sc as plsc

D_IN = 768
HIDDEN = 12288
K = 32
B = 8192
GROUP = 16
NGROUP = HIDDEN // GROUP

RB = 256
CB = 2048

NC = 2
NS = 16
NW = NC * NS
ROWS_PER_W = B // NW


def _enc_body(x_ref, w_ref, benc_ref, bdec_ref, acts_ref, summ_ref):
    xb = x_ref[...] - bdec_ref[...]
    acc = lax.dot_general(
        xb, w_ref[...],
        dimension_numbers=(((1,), (1,)), ((), ())),
        preferred_element_type=jnp.float32,
    )
    acts = jnp.maximum(acc + benc_ref[...], 0.0)
    acts_ref[...] = acts.reshape(RB // 8, 8, CB // 128, 128)
    summ_ref[...] = jnp.max(acts.reshape(RB, CB // GROUP, GROUP), axis=-1)


def _encode(x, W_enc, b_enc, b_dec):
    grid = (HIDDEN // CB, B // RB)
    return pl.pallas_call(
        _enc_body,
        grid=grid,
        in_specs=[
            pl.BlockSpec((RB, D_IN), lambda j, i: (i, 0)),
            pl.BlockSpec((CB, D_IN), lambda j, i: (j, 0)),
            pl.BlockSpec((1, CB), lambda j, i: (0, j)),
            pl.BlockSpec((1, D_IN), lambda j, i: (0, 0)),
        ],
        out_specs=[
            pl.BlockSpec((RB // 8, 8, CB // 128, 128),
                         lambda j, i: (i, 0, j, 0)),
            pl.BlockSpec((RB, CB // GROUP), lambda j, i: (i, j)),
        ],
        out_shape=[
            jax.ShapeDtypeStruct((B // 8, 8, HIDDEN // 128, 128), jnp.float32),
            jax.ShapeDtypeStruct((B, NGROUP), jnp.float32),
        ],
        compiler_params=pltpu.CompilerParams(
            dimension_semantics=("arbitrary", "arbitrary"),
        ),
    )(x, W_enc, b_enc.reshape(1, HIDDEN), b_dec.reshape(1, D_IN))


def _bcast(v, j):
    dnums = lax.GatherDimensionNumbers(
        offset_dims=(), collapsed_slice_dims=(0,), start_index_map=(0,))
    return lax.gather(v, jnp.full((16, 1), j, jnp.int32), dnums,
                      slice_sizes=(1,),
                      mode=lax.GatherScatterMode.PROMISE_IN_BOUNDS)


def _merge(state, ck, ci):
    t1k, t1i, t2k, t2i = state
    ckd, cid = plsc.sort_key_val(ck, ci, descending=True)
    m = t2k >= ckd
    h2k = jnp.where(m, t2k, ckd)
    h2i = jnp.where(m, t2i, cid)
    h2k, h2i = plsc.sort_key_val(h2k, h2i, descending=True)
    m2 = t1k >= h2k
    h1k = jnp.where(m2, t1k, h2k)
    h1i = jnp.where(m2, t1i, h2i)
    l1k = jnp.where(m2, h2k, t1k)
    l1i = jnp.where(m2, h2i, t1i)
    t1k, t1i = plsc.sort_key_val(h1k, h1i)
    t2k, t2i = plsc.sort_key_val(l1k, l1i)
    return (t1k, t1i, t2k, t2i)


_NEG = functools.partial(jnp.full, (16,), -1.0, jnp.float32)
_ZERO = functools.partial(jnp.zeros, (16,), jnp.int32)


def _sc_body(pa_ref, summ_ref, wdec_ref, bdec_ref, out_ref,
             s_v, cidx_v, cand_v, widx_v, wrows_v, bdec_v, acc_v,
             ssem0, ssem1, csem0, csem1, wsem0, wsem1, osem0, osem1):
    wid = lax.axis_index("s") * NC + lax.axis_index("c")
    row0 = wid * ROWS_PER_W
    pltpu.sync_copy(bdec_ref, bdec_v)
    lane = lax.iota(jnp.int32, 16)

    ssem = (ssem0, ssem1)
    csem = (csem0, csem1)
    wsem = (wsem0, wsem1)
    osem = (osem0, osem1)

    def summ_cp(r, p):
        return pltpu.make_async_copy(summ_ref.at[row0 + r], s_v.at[p], ssem[p])

    def cand_cp(r, p):
        return pltpu.make_async_copy(pa_ref.at[cidx_v.at[p]], cand_v.at[p],
                                     csem[p])

    def w_cp(p):
        return pltpu.make_async_copy(wdec_ref.at[widx_v.at[p]], wrows_v.at[p],
                                     wsem[p])

    def out_cp(r, p):
        return pltpu.make_async_copy(acc_v.at[p], out_ref.at[row0 + r],
                                     osem[p])

    def ph1(p, r):
        def step(g, st):
            stA = st[:4]
            stB = st[4:]
            gA = 2 * g
            gB = 2 * g + 1
            ckA = s_v[p, pl.ds(gA * GROUP, GROUP)]
            ckB = s_v[p, pl.ds(gB * GROUP, GROUP)]
            stA = _merge(stA, ckA, lane + gA * GROUP)
            stB = _merge(stB, ckB, lane + gB * GROUP)
            return stA + stB
        init = (_NEG(), _ZERO(), _NEG(), _ZERO()) * 2
        st = lax.fori_loop(0, NGROUP // GROUP // 2, step, init)
        stA = _merge(st[:4], st[4], st[5])
        stA = _merge(stA, st[6], st[7])
        _, g1, _, g2 = stA
        cidx_v[p, pl.ds(0, 16)] = g1 + (row0 + r) * NGROUP
        cidx_v[p, pl.ds(16, 16)] = g2 + (row0 + r) * NGROUP
        cand_cp(r, p).start()
        return g1, g2

    def ph2(p, g1, g2):
        def step(j, st):
            stA = st[:4]
            stB = st[4:]
            stA = _merge(stA, cand_v[p, j, pl.ds(0, GROUP)],
                         _bcast(g1, j) * GROUP + lane)
            stB = _merge(stB, cand_v[p, j + 16, pl.ds(0, GROUP)],
                         _bcast(g2, j) * GROUP + lane)
            return stA + stB
        init = (_NEG(), _ZERO(), _NEG(), _ZERO()) * 2
        st = lax.fori_loop(0, 16, step, init)
        stA = _merge(st[:4], st[4], st[5])
        stA = _merge(stA, st[6], st[7])
        v1, i1, v2, i2 = stA
        widx_v[p, pl.ds(0, 16)] = i1
        widx_v[p, pl.ds(16, 16)] = i2
        w_cp(p).start()
        return v1, v2

    def decode(p, v1, v2):
        for c in range(D_IN // 16):
            acc_v[p, pl.ds(c * 16, 16)] = bdec_v[pl.ds(c * 16, 16)]

        def kloop(k, vk):
            v = _bcast(vk, k)
            for c in range(D_IN // 16):
                plsc.addupdate(
                    acc_v.at[p, pl.ds(c * 16, 16)],
                    v * wrows_v[p, k, pl.ds(c * 16, 16)])
            return vk

        def kloop2(k, vk):
            v = _bcast(vk, k)
            for c in range(D_IN // 16):
                plsc.addupdate(
                    acc_v.at[p, pl.ds(c * 16, 16)],
                    v * wrows_v[p, k + 16, pl.ds(c * 16, 16)])
            return vk
        lax.fori_loop(0, 16, kloop, v1)
        lax.fori_loop(0, 16, kloop2, v2)

    summ_cp(0, 0).start()
    summ_cp(1, 1).start()
    summ_cp(0, 0).wait()
    g_carry = ph1(0, 0)
    summ_cp(2, 0).start()

    def pair(t, carry, do_next):
        a = 2 * t
        g1a, g2a = carry
        cand_cp(a, 0).wait()
        va1, va2 = ph2(0, g1a, g2a)
        summ_cp(a + 1, 1).wait()
        g1b, g2b = ph1(1, a + 1)
        if do_next:
            summ_cp(a + 3, 1).start()

        @pl.when(t > 0)
        def _():
            out_cp(a, 0).wait()
        w_cp(0).wait()
        decode(0, va1, va2)
        out_cp(a, 0).start()
        cand_cp(a + 1, 1).wait()
        vb1, vb2 = ph2(1, g1b, g2b)
        if do_next:
            summ_cp(a + 2, 0).wait()
            g_next = ph1(0, a + 2)
            summ_cp(jnp.minimum(a + 4, ROWS_PER_W - 1), 0).start()
        else:
            g_next = (_ZERO(), _ZERO())

        @pl.when(t > 0)
        def _():
            out_cp(a + 1, 1).wait()
        w_cp(1).wait()
        decode(1, vb1, vb2)
        out_cp(a + 1, 1).start()
        return g_next

    body = functools.partial(pair, do_next=True)
    g_carry = lax.fori_loop(0, ROWS_PER_W // 2 - 1, body, g_carry)
    pair(ROWS_PER_W // 2 - 1, g_carry, do_next=False)
    summ_cp(ROWS_PER_W - 1, 0).wait()
    out_cp(ROWS_PER_W - 2, 0).wait()
    out_cp(ROWS_PER_W - 1, 1).wait()


def _decode_topk(pre_acts4, summaries, W_dec, b_dec):
    pa_flat = pre_acts4.reshape(B * NGROUP, GROUP)
    mesh = plsc.VectorSubcoreMesh(core_axis_name="c", subcore_axis_name="s")
    f = pl.kernel(
        _sc_body,
        out_type=jax.ShapeDtypeStruct((B, D_IN), jnp.float32),
        mesh=mesh,
        scratch_types=[
            pltpu.VMEM((2, NGROUP), jnp.float32),
            pltpu.VMEM((2, K), jnp.int32),
            pltpu.VMEM((2, K, GROUP), jnp.float32),
            pltpu.VMEM((2, K), jnp.int32),
            pltpu.VMEM((2, K, D_IN), jnp.float32),
            pltpu.VMEM((D_IN,), jnp.float32),
            pltpu.VMEM((2, D_IN), jnp.float32),
        ] + [pltpu.SemaphoreType.DMA] * 8,
        compiler_params=pltpu.CompilerParams(
            needs_layout_passes=False, use_tc_tiling_on_sc=False),
    )
    return f(pa_flat, summaries, W_dec, b_dec)


def kernel(x, W_enc, b_enc, W_dec, b_dec):
    pre_acts4, summaries = _encode(x, W_enc, b_enc, b_dec)
    return _decode_topk(pre_acts4, summaries, W_dec, b_dec)


# speedup vs baseline: 6.6918x; 1.5516x over previous
import functools

import jax
import jax.numpy as jnp
from jax import lax
from jax.experimental import pallas as pl
from jax.experimental.pallas import tpu as pltpu
from jax.experimental.pallas import tpu_sc as plsc

D_IN = 768
HIDDEN = 12288
K = 32
B = 8192
GROUP = 16
NGROUP = HIDDEN // GROUP

RB = 256
CB = 2048

NC = 2
NS = 16
NW = NC * NS
ROWS_PER_W = B // NW


def _enc_body(x_ref, w_ref, benc_ref, bdec_ref, acts_ref, summ_ref):
    xb = x_ref[...] - bdec_ref[...]
    acc = lax.dot_general(
        xb, w_ref[...],
        dimension_numbers=(((1,), (1,)), ((), ())),
        preferred_element_type=jnp.float32,
    )
    acts = jnp.maximum(acc + benc_ref[...], 0.0)
    m = acts[:, 0:128]
    for t in range(CB // 128):
        blk = acts[:, t * 128:(t + 1) * 128]
        acts_ref[:, t, :, :] = blk.reshape(RB // 8, 8, 128)
        if t > 0:
            m = jnp.maximum(m, blk)
    summ_ref[...] = m


def _encode(x, W_enc, b_enc, b_dec):
    grid = (HIDDEN // CB, B // RB)
    return pl.pallas_call(
        _enc_body,
        grid=grid,
        in_specs=[
            pl.BlockSpec((RB, D_IN), lambda j, i: (i, 0)),
            pl.BlockSpec((CB, D_IN), lambda j, i: (j, 0)),
            pl.BlockSpec((1, CB), lambda j, i: (0, j)),
            pl.BlockSpec((1, D_IN), lambda j, i: (0, 0)),
        ],
        out_specs=[
            pl.BlockSpec((RB // 8, CB // 128, 8, 128),
                         lambda j, i: (i, j, 0, 0)),
            pl.BlockSpec((RB, CB // GROUP), lambda j, i: (i, j)),
        ],
        out_shape=[
            jax.ShapeDtypeStruct((B // 8, HIDDEN // 128, 8, 128),
                                 jnp.float32),
            jax.ShapeDtypeStruct((B, NGROUP), jnp.float32),
        ],
        compiler_params=pltpu.CompilerParams(
            dimension_semantics=("arbitrary", "arbitrary"),
        ),
    )(x, W_enc, b_enc.reshape(1, HIDDEN), b_dec.reshape(1, D_IN))


def _bcast(v, j):
    dnums = lax.GatherDimensionNumbers(
        offset_dims=(), collapsed_slice_dims=(0,), start_index_map=(0,))
    return lax.gather(v, jnp.full((16, 1), j, jnp.int32), dnums,
                      slice_sizes=(1,),
                      mode=lax.GatherScatterMode.PROMISE_IN_BOUNDS)


def _merge(state, ck, ci):
    t1k, t1i, t2k, t2i = state
    ckd, cid = plsc.sort_key_val(ck, ci, descending=True)
    m = t2k >= ckd
    h2k = jnp.where(m, t2k, ckd)
    h2i = jnp.where(m, t2i, cid)
    h2k, h2i = plsc.sort_key_val(h2k, h2i, descending=True)
    m2 = t1k >= h2k
    h1k = jnp.where(m2, t1k, h2k)
    h1i = jnp.where(m2, t1i, h2i)
    l1k = jnp.where(m2, h2k, t1k)
    l1i = jnp.where(m2, h2i, t1i)
    t1k, t1i = plsc.sort_key_val(h1k, h1i)
    t2k, t2i = plsc.sort_key_val(l1k, l1i)
    return (t1k, t1i, t2k, t2i)


def _cx(ak, ai, bk, bi):
    m = ak >= bk
    hk = jnp.where(m, ak, bk)
    hi = jnp.where(m, ai, bi)
    lk = jnp.where(m, bk, ak)
    li = jnp.where(m, bi, ai)
    return hk, hi, lk, li


def _merge2(state, ck1, ci1, ck2, ci2):
    t1k, t1i, t2k, t2i = state
    c1k, c1i = plsc.sort_key_val(ck1, ci1)
    c2k, c2i = plsc.sort_key_val(ck2, ci2, descending=True)
    hk, hi, lk, li = _cx(c1k, c1i, c2k, c2i)
    b1k, b1i = plsc.sort_key_val(hk, hi, descending=True)
    b2k, b2i = plsc.sort_key_val(lk, li, descending=True)
    u1k, u1i, _, _ = _cx(t1k, t1i, b1k, b1i)
    u2k, u2i, _, _ = _cx(t2k, t2i, b2k, b2i)
    w2k, w2i, w1k, w1i = _cx(u1k, u1i, u2k, u2i)
    t1k, t1i = plsc.sort_key_val(w2k, w2i)
    t2k, t2i = plsc.sort_key_val(w1k, w1i)
    return (t1k, t1i, t2k, t2i)

_NEG = functools.partial(jnp.full, (16,), -1.0, jnp.float32)
_ZERO = functools.partial(jnp.zeros, (16,), jnp.int32)


def _sc_body(pa_ref, summ_ref, wdec_ref, bdec_ref, out_ref,
             s_v, cidx_v, cand_v, widx_v, wrows_v, bdec_v, acc_v,
             ssem0, ssem1, csem0, csem1, wsem0, wsem1, osem0, osem1):
    wid = lax.axis_index("s") * NC + lax.axis_index("c")
    row0 = wid * ROWS_PER_W
    pltpu.sync_copy(bdec_ref, bdec_v)
    lane = lax.iota(jnp.int32, 16)

    ssem = (ssem0, ssem1)
    csem = (csem0, csem1)
    wsem = (wsem0, wsem1)
    osem = (osem0, osem1)

    def summ_cp(r, p):
        return pltpu.make_async_copy(summ_ref.at[row0 + r], s_v.at[p], ssem[p])

    def cand_cp(r, p, q):
        return pltpu.make_async_copy(
            pa_ref.at[cidx_v.at[p, pl.ds(q * 128, 128)]],
            cand_v.at[p, pl.ds(q * 128, 128)], csem[p])

    def w_cp(p):
        return pltpu.make_async_copy(wdec_ref.at[widx_v.at[p]], wrows_v.at[p],
                                     wsem[p])

    def out_cp(r, p):
        return pltpu.make_async_copy(acc_v.at[p], out_ref.at[row0 + r],
                                     osem[p])

    def ph1(p, r):
        def step(g, st):
            out = []
            for ch in range(2):
                g0 = 4 * g + 2 * ch
                ck1 = s_v[p, pl.ds(g0 * GROUP, GROUP)]
                ck2 = s_v[p, pl.ds((g0 + 1) * GROUP, GROUP)]
                out.extend(_merge2(st[4 * ch:4 * ch + 4],
                                   ck1, lane + g0 * GROUP,
                                   ck2, lane + (g0 + 1) * GROUP))
            return tuple(out)
        init = (_NEG(), _ZERO(), _NEG(), _ZERO()) * 2
        st = lax.fori_loop(0, NGROUP // GROUP // 4, step, init)
        stA = _merge2(st[:4], st[4], st[5], st[6], st[7])
        _, g1, _, g2 = stA
        c1 = lax.shift_right_logical(g1, 7) * CB + (g1 & 127)
        c2 = lax.shift_right_logical(g2, 7) * CB + (g2 & 127)
        row = row0 + r
        tbase = ((row // 8) * (HIDDEN // 128)) * 1024 + (row % 8) * 128
        b1 = tbase + lax.shift_right_logical(g1, 7) * (16 * 1024) + (g1 & 127)
        b2 = tbase + lax.shift_right_logical(g2, 7) * (16 * 1024) + (g2 & 127)
        mstep = lane * 1024
        for j in range(16):
            cidx_v[p, pl.ds(j * 16, 16)] = _bcast(b1, j) + mstep
            cidx_v[p, pl.ds(256 + j * 16, 16)] = _bcast(b2, j) + mstep
        for q in range(4):
            cand_cp(r, p, q).start()
        return c1, c2

    def ph2(p, c1, c2):
        mstep = lane * 128
        def step(j, st):
            stA = st[:4]
            stB = st[4:]
            j0 = 2 * j
            stA = _merge2(stA,
                          cand_v[p, pl.ds(j0 * 16, GROUP)],
                          _bcast(c1, j0) + mstep,
                          cand_v[p, pl.ds((j0 + 1) * 16, GROUP)],
                          _bcast(c1, j0 + 1) + mstep)
            stB = _merge2(stB,
                          cand_v[p, pl.ds(256 + j0 * 16, GROUP)],
                          _bcast(c2, j0) + mstep,
                          cand_v[p, pl.ds(256 + (j0 + 1) * 16, GROUP)],
                          _bcast(c2, j0 + 1) + mstep)
            return stA + stB
        init = (_NEG(), _ZERO(), _NEG(), _ZERO()) * 2
        st = lax.fori_loop(0, 8, step, init)
        stA = _merge2(st[:4], st[4], st[5], st[6], st[7])
        v1, i1, v2, i2 = stA
        widx_v[p, pl.ds(0, 16)] = i1
        widx_v[p, pl.ds(16, 16)] = i2
        w_cp(p).start()
        return v1, v2

    def decode(p, v1, v2):
        for c in range(D_IN // 16):
            acc_v[p, pl.ds(c * 16, 16)] = bdec_v[pl.ds(c * 16, 16)]

        def kloop(k, vk):
            v = _bcast(vk, k)
            for c in range(D_IN // 16):
                plsc.addupdate(
                    acc_v.at[p, pl.ds(c * 16, 16)],
                    v * wrows_v[p, k, pl.ds(c * 16, 16)])
            return vk

        def kloop2(k, vk):
            v = _bcast(vk, k)
            for c in range(D_IN // 16):
                plsc.addupdate(
                    acc_v.at[p, pl.ds(c * 16, 16)],
                    v * wrows_v[p, k + 16, pl.ds(c * 16, 16)])
            return vk
        lax.fori_loop(0, 16, kloop, v1)
        lax.fori_loop(0, 16, kloop2, v2)

    summ_cp(0, 0).start()
    summ_cp(1, 1).start()
    summ_cp(0, 0).wait()
    g_carry = ph1(0, 0)
    summ_cp(2, 0).start()

    def pair(t, carry, do_next):
        a = 2 * t
        g1a, g2a = carry
        for q in range(4):
            cand_cp(a, 0, q).wait()
        va1, va2 = ph2(0, g1a, g2a)
        summ_cp(a + 1, 1).wait()
        g1b, g2b = ph1(1, a + 1)
        if do_next:
            summ_cp(a + 3, 1).start()

        @pl.when(t > 0)
        def _():
            out_cp(a, 0).wait()
        w_cp(0).wait()
        decode(0, va1, va2)
        out_cp(a, 0).start()
        for q in range(4):
            cand_cp(a + 1, 1, q).wait()
        vb1, vb2 = ph2(1, g1b, g2b)
        if do_next:
            summ_cp(a + 2, 0).wait()
            g_next = ph1(0, a + 2)
            summ_cp(jnp.minimum(a + 4, ROWS_PER_W - 1), 0).start()
        else:
            g_next = (_ZERO(), _ZERO())

        @pl.when(t > 0)
        def _():
            out_cp(a + 1, 1).wait()
        w_cp(1).wait()
        decode(1, vb1, vb2)
        out_cp(a + 1, 1).start()
        return g_next

    body = functools.partial(pair, do_next=True)
    g_carry = lax.fori_loop(0, ROWS_PER_W // 2 - 1, body, g_carry)
    pair(ROWS_PER_W // 2 - 1, g_carry, do_next=False)
    summ_cp(ROWS_PER_W - 1, 0).wait()
    out_cp(ROWS_PER_W - 2, 0).wait()
    out_cp(ROWS_PER_W - 1, 1).wait()


def _decode_topk(pre_acts, summaries, W_dec, b_dec):
    pa_flat = pre_acts.reshape(B * HIDDEN)
    mesh = plsc.VectorSubcoreMesh(core_axis_name="c", subcore_axis_name="s")
    f = pl.kernel(
        _sc_body,
        out_type=jax.ShapeDtypeStruct((B, D_IN), jnp.float32),
        mesh=mesh,
        scratch_types=[
            pltpu.VMEM((2, NGROUP), jnp.float32),
            pltpu.VMEM((2, K * GROUP), jnp.int32),
            pltpu.VMEM((2, K * GROUP), jnp.float32),
            pltpu.VMEM((2, K), jnp.int32),
            pltpu.VMEM((2, K, D_IN), jnp.float32),
            pltpu.VMEM((D_IN,), jnp.float32),
            pltpu.VMEM((2, D_IN), jnp.float32),
        ] + [pltpu.SemaphoreType.DMA] * 8,
        compiler_params=pltpu.CompilerParams(
            needs_layout_passes=False, use_tc_tiling_on_sc=False),
    )
    return f(pa_flat, summaries, W_dec, b_dec)


def kernel(x, W_enc, b_enc, W_dec, b_dec):
    pre_acts, summaries = _encode(x, W_enc, b_enc, b_dec)
    return _decode_topk(pre_acts, summaries, W_dec, b_dec)
